# baseline (device time: 98957 ns/iter reference)
import jax
import jax.numpy as jnp
from jax import lax
from jax.experimental import pallas as pl
from jax.experimental.pallas import tpu as pltpu

NC = 4
QROWS = 1024
NDIAG = NC // 2


def kernel(x):
    m, n = x.shape
    r = QROWS // NC

    def body(x_ref, out_ref, yrecv, xrecv_sh, xrecv_dg, zrecv_sh, zrecv_dg,
             ysend_sem, yrecv_sem, xsend_sem, xrecv_sh_sem, xrecv_dg_sem,
             zsend_sem, zrecv_sh_sem, zrecv_dg_sem):
        my_x = lax.axis_index("x")
        my_y = lax.axis_index("y")
        my_z = lax.axis_index("z")
        p = my_z % 2
        prt_z = my_z + 1 - 2 * p
        y_nbr = (my_x, 1 - my_y, my_z)
        x_nbr = (1 - my_x, my_y, my_z)
        z_nbr = (my_x, my_y, prt_z)

        a_base = my_x * 2048 + p * QROWS
        xsh_base = (1 - my_x) * 2048 + p * QROWS
        zsh_base = my_x * 2048 + (1 - p) * QROWS
        dg_base = (1 - my_x) * 2048 + (1 - p) * QROWS

        barrier_sem = pltpu.get_barrier_semaphore()
        for nbr in (y_nbr, x_nbr, z_nbr):
            pl.semaphore_signal(
                barrier_sem, inc=1,
                device_id=nbr, device_id_type=pl.DeviceIdType.MESH,
            )
        pl.semaphore_wait(barrier_sem, 3)

        y_rdmas = []
        for i in range(NC):
            rdma = pltpu.make_async_remote_copy(
                src_ref=x_ref.at[pl.ds(a_base + i * r, r), :],
                dst_ref=yrecv.at[i],
                send_sem=ysend_sem.at[i],
                recv_sem=yrecv_sem.at[i],
                device_id=y_nbr,
                device_id_type=pl.DeviceIdType.MESH,
            )
            rdma.start()
            y_rdmas.append(rdma)

        x_rdmas = []
        z_rdmas = []

        for i in range(NC):
            y_rdmas[i].wait_recv()
            fx = pltpu.make_async_remote_copy(
                src_ref=yrecv.at[i],
                dst_ref=xrecv_sh.at[i],
                send_sem=xsend_sem.at[i],
                recv_sem=xrecv_sh_sem.at[i],
                device_id=x_nbr,
                device_id_type=pl.DeviceIdType.MESH,
            )
            fx.start()
            x_rdmas.append(fx)
            fz = pltpu.make_async_remote_copy(
                src_ref=yrecv.at[i],
                dst_ref=zrecv_sh.at[i],
                send_sem=zsend_sem.at[i],
                recv_sem=zrecv_sh_sem.at[i],
                device_id=z_nbr,
                device_id_type=pl.DeviceIdType.MESH,
            )
            fz.start()
            z_rdmas.append(fz)
            rows = pl.ds(a_base + i * r, r)
            out_ref[rows, :] = x_ref[rows, :] + yrecv[i, :, :]

        for i in range(NC):
            zrecv_sh_rdma = pltpu.make_async_remote_copy(
                src_ref=yrecv.at[0],
                dst_ref=zrecv_sh.at[i],
                send_sem=ysend_sem.at[0],
                recv_sem=zrecv_sh_sem.at[i],
                device_id=z_nbr,
                device_id_type=pl.DeviceIdType.MESH,
            )
            zrecv_sh_rdma.wait_recv()
            if i < NDIAG:
                fd = pltpu.make_async_remote_copy(
                    src_ref=zrecv_sh.at[i],
                    dst_ref=xrecv_dg.at[i],
                    send_sem=xsend_sem.at[NC + i],
                    recv_sem=xrecv_dg_sem.at[i],
                    device_id=x_nbr,
                    device_id_type=pl.DeviceIdType.MESH,
                )
                fd.start()
                x_rdmas.append(fd)
            rows = pl.ds(zsh_base + i * r, r)
            out_ref[rows, :] = x_ref[rows, :] + zrecv_sh[i, :, :]

        for i in range(NC):
            xrecv_sh_rdma = pltpu.make_async_remote_copy(
                src_ref=yrecv.at[0],
                dst_ref=xrecv_sh.at[i],
                send_sem=ysend_sem.at[0],
                recv_sem=xrecv_sh_sem.at[i],
                device_id=x_nbr,
                device_id_type=pl.DeviceIdType.MESH,
            )
            xrecv_sh_rdma.wait_recv()
            if i >= NDIAG:
                fd = pltpu.make_async_remote_copy(
                    src_ref=xrecv_sh.at[i],
                    dst_ref=zrecv_dg.at[i - NDIAG],
                    send_sem=zsend_sem.at[NC + i - NDIAG],
                    recv_sem=zrecv_dg_sem.at[i - NDIAG],
                    device_id=z_nbr,
                    device_id_type=pl.DeviceIdType.MESH,
                )
                fd.start()
                z_rdmas.append(fd)
            rows = pl.ds(xsh_base + i * r, r)
            out_ref[rows, :] = x_ref[rows, :] + xrecv_sh[i, :, :]

        for i in range(NDIAG):
            dg = pltpu.make_async_remote_copy(
                src_ref=yrecv.at[0],
                dst_ref=xrecv_dg.at[i],
                send_sem=ysend_sem.at[0],
                recv_sem=xrecv_dg_sem.at[i],
                device_id=x_nbr,
                device_id_type=pl.DeviceIdType.MESH,
            )
            dg.wait_recv()
            rows = pl.ds(dg_base + i * r, r)
            out_ref[rows, :] = x_ref[rows, :] + xrecv_dg[i, :, :]
        for i in range(NDIAG):
            dg = pltpu.make_async_remote_copy(
                src_ref=yrecv.at[0],
                dst_ref=zrecv_dg.at[i],
                send_sem=ysend_sem.at[0],
                recv_sem=zrecv_dg_sem.at[i],
                device_id=z_nbr,
                device_id_type=pl.DeviceIdType.MESH,
            )
            dg.wait_recv()
            rows = pl.ds(dg_base + (NDIAG + i) * r, r)
            out_ref[rows, :] = x_ref[rows, :] + zrecv_dg[i, :, :]

        for rdma in y_rdmas:
            rdma.wait_send()
        for rdma in x_rdmas:
            rdma.wait_send()
        for rdma in z_rdmas:
            rdma.wait_send()

    return pl.pallas_call(
        body,
        out_shape=jax.ShapeDtypeStruct((m, n), x.dtype),
        in_specs=[pl.BlockSpec(memory_space=pltpu.VMEM)],
        out_specs=pl.BlockSpec(memory_space=pltpu.VMEM),
        scratch_shapes=[
            pltpu.VMEM((NC, r, n), x.dtype),
            pltpu.VMEM((NC, r, n), x.dtype),
            pltpu.VMEM((NDIAG, r, n), x.dtype),
            pltpu.VMEM((NC, r, n), x.dtype),
            pltpu.VMEM((NDIAG, r, n), x.dtype),
            pltpu.SemaphoreType.DMA((NC,)),
            pltpu.SemaphoreType.DMA((NC,)),
            pltpu.SemaphoreType.DMA((NC + NDIAG,)),
            pltpu.SemaphoreType.DMA((NC,)),
            pltpu.SemaphoreType.DMA((NDIAG,)),
            pltpu.SemaphoreType.DMA((NC + NDIAG,)),
            pltpu.SemaphoreType.DMA((NC,)),
            pltpu.SemaphoreType.DMA((NDIAG,)),
        ],
        compiler_params=pltpu.CompilerParams(collective_id=0),
    )(x)


# device time: 91267 ns/iter; 1.0843x vs baseline; 1.0843x over previous
import jax
import jax.numpy as jnp
from jax import lax
from jax.experimental import pallas as pl
from jax.experimental.pallas import tpu as pltpu

NC = 16
QROWS = 1024
NDIAG = NC // 2


def kernel(x):
    m, n = x.shape
    r = QROWS // NC

    def body(x_ref, out_ref, yrecv, xrecv_sh, xrecv_dg, zrecv_sh, zrecv_dg,
             ysend_sem, yrecv_sem, xsend_sem, xrecv_sh_sem, xrecv_dg_sem,
             zsend_sem, zrecv_sh_sem, zrecv_dg_sem):
        my_x = lax.axis_index("x")
        my_y = lax.axis_index("y")
        my_z = lax.axis_index("z")
        p = my_z % 2
        prt_z = my_z + 1 - 2 * p
        y_nbr = (my_x, 1 - my_y, my_z)
        x_nbr = (1 - my_x, my_y, my_z)
        z_nbr = (my_x, my_y, prt_z)

        a_base = my_x * 2048 + p * QROWS
        xsh_base = (1 - my_x) * 2048 + p * QROWS
        zsh_base = my_x * 2048 + (1 - p) * QROWS
        dg_base = (1 - my_x) * 2048 + (1 - p) * QROWS

        barrier_sem = pltpu.get_barrier_semaphore()
        for nbr in (y_nbr, x_nbr, z_nbr):
            pl.semaphore_signal(
                barrier_sem, inc=1,
                device_id=nbr, device_id_type=pl.DeviceIdType.MESH,
            )
        pl.semaphore_wait(barrier_sem, 3)

        y_rdmas = []
        for i in range(NC):
            rdma = pltpu.make_async_remote_copy(
                src_ref=x_ref.at[pl.ds(a_base + i * r, r), :],
                dst_ref=yrecv.at[i],
                send_sem=ysend_sem.at[i],
                recv_sem=yrecv_sem.at[i],
                device_id=y_nbr,
                device_id_type=pl.DeviceIdType.MESH,
            )
            rdma.start()
            y_rdmas.append(rdma)

        x_rdmas = []
        z_rdmas = []

        for i in range(NC):
            y_rdmas[i].wait_recv()
            fx = pltpu.make_async_remote_copy(
                src_ref=yrecv.at[i],
                dst_ref=xrecv_sh.at[i],
                send_sem=xsend_sem.at[i],
                recv_sem=xrecv_sh_sem.at[i],
                device_id=x_nbr,
                device_id_type=pl.DeviceIdType.MESH,
            )
            fx.start()
            x_rdmas.append(fx)
            fz = pltpu.make_async_remote_copy(
                src_ref=yrecv.at[i],
                dst_ref=zrecv_sh.at[i],
                send_sem=zsend_sem.at[i],
                recv_sem=zrecv_sh_sem.at[i],
                device_id=z_nbr,
                device_id_type=pl.DeviceIdType.MESH,
            )
            fz.start()
            z_rdmas.append(fz)
            rows = pl.ds(a_base + i * r, r)
            out_ref[rows, :] = x_ref[rows, :] + yrecv[i, :, :]

        for i in range(NC):
            zrecv_sh_rdma = pltpu.make_async_remote_copy(
                src_ref=yrecv.at[0],
                dst_ref=zrecv_sh.at[i],
                send_sem=ysend_sem.at[0],
                recv_sem=zrecv_sh_sem.at[i],
                device_id=z_nbr,
                device_id_type=pl.DeviceIdType.MESH,
            )
            zrecv_sh_rdma.wait_recv()
            if i < NDIAG:
                fd = pltpu.make_async_remote_copy(
                    src_ref=zrecv_sh.at[i],
                    dst_ref=xrecv_dg.at[i],
                    send_sem=xsend_sem.at[NC + i],
                    recv_sem=xrecv_dg_sem.at[i],
                    device_id=x_nbr,
                    device_id_type=pl.DeviceIdType.MESH,
                )
                fd.start()
                x_rdmas.append(fd)
            rows = pl.ds(zsh_base + i * r, r)
            out_ref[rows, :] = x_ref[rows, :] + zrecv_sh[i, :, :]

        for i in range(NC):
            xrecv_sh_rdma = pltpu.make_async_remote_copy(
                src_ref=yrecv.at[0],
                dst_ref=xrecv_sh.at[i],
                send_sem=ysend_sem.at[0],
                recv_sem=xrecv_sh_sem.at[i],
                device_id=x_nbr,
                device_id_type=pl.DeviceIdType.MESH,
            )
            xrecv_sh_rdma.wait_recv()
            if i >= NDIAG:
                fd = pltpu.make_async_remote_copy(
                    src_ref=xrecv_sh.at[i],
                    dst_ref=zrecv_dg.at[i - NDIAG],
                    send_sem=zsend_sem.at[NC + i - NDIAG],
                    recv_sem=zrecv_dg_sem.at[i - NDIAG],
                    device_id=z_nbr,
                    device_id_type=pl.DeviceIdType.MESH,
                )
                fd.start()
                z_rdmas.append(fd)
            rows = pl.ds(xsh_base + i * r, r)
            out_ref[rows, :] = x_ref[rows, :] + xrecv_sh[i, :, :]

        for i in range(NDIAG):
            dg = pltpu.make_async_remote_copy(
                src_ref=yrecv.at[0],
                dst_ref=xrecv_dg.at[i],
                send_sem=ysend_sem.at[0],
                recv_sem=xrecv_dg_sem.at[i],
                device_id=x_nbr,
                device_id_type=pl.DeviceIdType.MESH,
            )
            dg.wait_recv()
            rows = pl.ds(dg_base + i * r, r)
            out_ref[rows, :] = x_ref[rows, :] + xrecv_dg[i, :, :]
        for i in range(NDIAG):
            dg = pltpu.make_async_remote_copy(
                src_ref=yrecv.at[0],
                dst_ref=zrecv_dg.at[i],
                send_sem=ysend_sem.at[0],
                recv_sem=zrecv_dg_sem.at[i],
                device_id=z_nbr,
                device_id_type=pl.DeviceIdType.MESH,
            )
            dg.wait_recv()
            rows = pl.ds(dg_base + (NDIAG + i) * r, r)
            out_ref[rows, :] = x_ref[rows, :] + zrecv_dg[i, :, :]

        for rdma in y_rdmas:
            rdma.wait_send()
        for rdma in x_rdmas:
            rdma.wait_send()
        for rdma in z_rdmas:
            rdma.wait_send()

    return pl.pallas_call(
        body,
        out_shape=jax.ShapeDtypeStruct((m, n), x.dtype),
        in_specs=[pl.BlockSpec(memory_space=pltpu.VMEM)],
        out_specs=pl.BlockSpec(memory_space=pltpu.VMEM),
        scratch_shapes=[
            pltpu.VMEM((NC, r, n), x.dtype),
            pltpu.VMEM((NC, r, n), x.dtype),
            pltpu.VMEM((NDIAG, r, n), x.dtype),
            pltpu.VMEM((NC, r, n), x.dtype),
            pltpu.VMEM((NDIAG, r, n), x.dtype),
            pltpu.SemaphoreType.DMA((NC,)),
            pltpu.SemaphoreType.DMA((NC,)),
            pltpu.SemaphoreType.DMA((NC + NDIAG,)),
            pltpu.SemaphoreType.DMA((NC,)),
            pltpu.SemaphoreType.DMA((NDIAG,)),
            pltpu.SemaphoreType.DMA((NC + NDIAG,)),
            pltpu.SemaphoreType.DMA((NC,)),
            pltpu.SemaphoreType.DMA((NDIAG,)),
        ],
        compiler_params=pltpu.CompilerParams(collective_id=0),
    )(x)


# device time: 89792 ns/iter; 1.1021x vs baseline; 1.0164x over previous
import jax
import jax.numpy as jnp
from jax import lax
from jax.experimental import pallas as pl
from jax.experimental.pallas import tpu as pltpu

NC = 32
QROWS = 1024
NDIAG = NC // 2


def kernel(x):
    m, n = x.shape
    r = QROWS // NC

    def body(x_ref, out_ref, yrecv, xrecv_sh, xrecv_dg, zrecv_sh, zrecv_dg,
             ysend_sem, yrecv_sem, xsend_sem, xrecv_sh_sem, xrecv_dg_sem,
             zsend_sem, zrecv_sh_sem, zrecv_dg_sem):
        my_x = lax.axis_index("x")
        my_y = lax.axis_index("y")
        my_z = lax.axis_index("z")
        p = my_z % 2
        prt_z = my_z + 1 - 2 * p
        y_nbr = (my_x, 1 - my_y, my_z)
        x_nbr = (1 - my_x, my_y, my_z)
        z_nbr = (my_x, my_y, prt_z)

        a_base = my_x * 2048 + p * QROWS
        xsh_base = (1 - my_x) * 2048 + p * QROWS
        zsh_base = my_x * 2048 + (1 - p) * QROWS
        dg_base = (1 - my_x) * 2048 + (1 - p) * QROWS

        barrier_sem = pltpu.get_barrier_semaphore()
        for nbr in (y_nbr, x_nbr, z_nbr):
            pl.semaphore_signal(
                barrier_sem, inc=1,
                device_id=nbr, device_id_type=pl.DeviceIdType.MESH,
            )
        pl.semaphore_wait(barrier_sem, 3)

        y_rdmas = []
        for i in range(NC):
            rdma = pltpu.make_async_remote_copy(
                src_ref=x_ref.at[pl.ds(a_base + i * r, r), :],
                dst_ref=yrecv.at[i],
                send_sem=ysend_sem.at[i],
                recv_sem=yrecv_sem.at[i],
                device_id=y_nbr,
                device_id_type=pl.DeviceIdType.MESH,
            )
            rdma.start()
            y_rdmas.append(rdma)

        x_rdmas = []
        z_rdmas = []

        for i in range(NC):
            y_rdmas[i].wait_recv()
            fx = pltpu.make_async_remote_copy(
                src_ref=yrecv.at[i],
                dst_ref=xrecv_sh.at[i],
                send_sem=xsend_sem.at[i],
                recv_sem=xrecv_sh_sem.at[i],
                device_id=x_nbr,
                device_id_type=pl.DeviceIdType.MESH,
            )
            fx.start()
            x_rdmas.append(fx)
            fz = pltpu.make_async_remote_copy(
                src_ref=yrecv.at[i],
                dst_ref=zrecv_sh.at[i],
                send_sem=zsend_sem.at[i],
                recv_sem=zrecv_sh_sem.at[i],
                device_id=z_nbr,
                device_id_type=pl.DeviceIdType.MESH,
            )
            fz.start()
            z_rdmas.append(fz)
            rows = pl.ds(a_base + i * r, r)
            out_ref[rows, :] = x_ref[rows, :] + yrecv[i, :, :]

        for i in range(NC):
            zrecv_sh_rdma = pltpu.make_async_remote_copy(
                src_ref=yrecv.at[0],
                dst_ref=zrecv_sh.at[i],
                send_sem=ysend_sem.at[0],
                recv_sem=zrecv_sh_sem.at[i],
                device_id=z_nbr,
                device_id_type=pl.DeviceIdType.MESH,
            )
            zrecv_sh_rdma.wait_recv()
            if i < NDIAG:
                fd = pltpu.make_async_remote_copy(
                    src_ref=zrecv_sh.at[i],
                    dst_ref=xrecv_dg.at[i],
                    send_sem=xsend_sem.at[NC + i],
                    recv_sem=xrecv_dg_sem.at[i],
                    device_id=x_nbr,
                    device_id_type=pl.DeviceIdType.MESH,
                )
                fd.start()
                x_rdmas.append(fd)
            rows = pl.ds(zsh_base + i * r, r)
            out_ref[rows, :] = x_ref[rows, :] + zrecv_sh[i, :, :]

        for i in range(NC):
            xrecv_sh_rdma = pltpu.make_async_remote_copy(
                src_ref=yrecv.at[0],
                dst_ref=xrecv_sh.at[i],
                send_sem=ysend_sem.at[0],
                recv_sem=xrecv_sh_sem.at[i],
                device_id=x_nbr,
                device_id_type=pl.DeviceIdType.MESH,
            )
            xrecv_sh_rdma.wait_recv()
            if i >= NDIAG:
                fd = pltpu.make_async_remote_copy(
                    src_ref=xrecv_sh.at[i],
                    dst_ref=zrecv_dg.at[i - NDIAG],
                    send_sem=zsend_sem.at[NC + i - NDIAG],
                    recv_sem=zrecv_dg_sem.at[i - NDIAG],
                    device_id=z_nbr,
                    device_id_type=pl.DeviceIdType.MESH,
                )
                fd.start()
                z_rdmas.append(fd)
            rows = pl.ds(xsh_base + i * r, r)
            out_ref[rows, :] = x_ref[rows, :] + xrecv_sh[i, :, :]

        for i in range(NDIAG):
            dg = pltpu.make_async_remote_copy(
                src_ref=yrecv.at[0],
                dst_ref=xrecv_dg.at[i],
                send_sem=ysend_sem.at[0],
                recv_sem=xrecv_dg_sem.at[i],
                device_id=x_nbr,
                device_id_type=pl.DeviceIdType.MESH,
            )
            dg.wait_recv()
            rows = pl.ds(dg_base + i * r, r)
            out_ref[rows, :] = x_ref[rows, :] + xrecv_dg[i, :, :]
        for i in range(NDIAG):
            dg = pltpu.make_async_remote_copy(
                src_ref=yrecv.at[0],
                dst_ref=zrecv_dg.at[i],
                send_sem=ysend_sem.at[0],
                recv_sem=zrecv_dg_sem.at[i],
                device_id=z_nbr,
                device_id_type=pl.DeviceIdType.MESH,
            )
            dg.wait_recv()
            rows = pl.ds(dg_base + (NDIAG + i) * r, r)
            out_ref[rows, :] = x_ref[rows, :] + zrecv_dg[i, :, :]

        for rdma in y_rdmas:
            rdma.wait_send()
        for rdma in x_rdmas:
            rdma.wait_send()
        for rdma in z_rdmas:
            rdma.wait_send()

    return pl.pallas_call(
        body,
        out_shape=jax.ShapeDtypeStruct((m, n), x.dtype),
        in_specs=[pl.BlockSpec(memory_space=pltpu.VMEM)],
        out_specs=pl.BlockSpec(memory_space=pltpu.VMEM),
        scratch_shapes=[
            pltpu.VMEM((NC, r, n), x.dtype),
            pltpu.VMEM((NC, r, n), x.dtype),
            pltpu.VMEM((NDIAG, r, n), x.dtype),
            pltpu.VMEM((NC, r, n), x.dtype),
            pltpu.VMEM((NDIAG, r, n), x.dtype),
            pltpu.SemaphoreType.DMA((NC,)),
            pltpu.SemaphoreType.DMA((NC,)),
            pltpu.SemaphoreType.DMA((NC + NDIAG,)),
            pltpu.SemaphoreType.DMA((NC,)),
            pltpu.SemaphoreType.DMA((NDIAG,)),
            pltpu.SemaphoreType.DMA((NC + NDIAG,)),
            pltpu.SemaphoreType.DMA((NC,)),
            pltpu.SemaphoreType.DMA((NDIAG,)),
        ],
        compiler_params=pltpu.CompilerParams(collective_id=0),
    )(x)
